# baseline (speedup 1.0000x reference)
import functools

import jax
import jax.numpy as jnp
from jax import lax
from jax.experimental import pallas as pl
from jax.experimental.pallas import tpu as pltpu
from jax.experimental.pallas import tpu_sc as plsc

N = 16384
DIM = 32
REG_USER = 0.0025
REG_POS_ITEM = 0.0025
REG_NEG_ITEM = 0.00025
REG_BIAS = 0.001

_INFO = plsc.get_sparse_core_info()
_NC = _INFO.num_cores
_NS = _INFO.num_subcores
_NW = _NC * _NS
_BPW = N // _NW
_L = 16


def _sc_body(u_hbm, i_hbm, j_hbm, w_hbm, h_hbm, b_hbm,
             x_hbm, swu_hbm, shi_hbm, shj_hbm, bs_hbm,
             u_v, i_v, j_v, wu_v, hi_v, hj_v, bi_v, bj_v,
             x_v, swu_v, shi_v, shj_v, bs_v, sem):
    wid = lax.axis_index("s") * _NC + lax.axis_index("c")
    base = wid * _BPW

    pltpu.sync_copy(u_hbm.at[pl.ds(base, _BPW)], u_v)
    pltpu.sync_copy(i_hbm.at[pl.ds(base, _BPW)], i_v)
    pltpu.sync_copy(j_hbm.at[pl.ds(base, _BPW)], j_v)

    c1 = pltpu.async_copy(w_hbm.at[u_v], wu_v, sem)
    c2 = pltpu.async_copy(h_hbm.at[i_v], hi_v, sem)
    c3 = pltpu.async_copy(h_hbm.at[j_v], hj_v, sem)
    c4 = pltpu.async_copy(b_hbm.at[i_v], bi_v, sem)
    c5 = pltpu.async_copy(b_hbm.at[j_v], bj_v, sem)
    c1.wait()
    c2.wait()
    c3.wait()
    c4.wait()
    c5.wait()

    iota = lax.iota(jnp.int32, _L)

    def block(blk, _):
        r0 = blk * _L
        rows = r0 + iota
        zero = jnp.zeros((_L,), jnp.float32)
        acc_ui = zero
        acc_uj = zero
        acc_wu = zero
        acc_hi = zero
        acc_hj = zero
        for d in range(DIM):
            col = jnp.full((_L,), d, jnp.int32)
            cw = plsc.load_gather(wu_v, [rows, col])
            ci = plsc.load_gather(hi_v, [rows, col])
            cj = plsc.load_gather(hj_v, [rows, col])
            acc_ui = acc_ui + cw * ci
            acc_uj = acc_uj + cw * cj
            acc_wu = acc_wu + cw * cw
            acc_hi = acc_hi + ci * ci
            acc_hj = acc_hj + cj * cj
        bi = bi_v[pl.ds(r0, _L)]
        bj = bj_v[pl.ds(r0, _L)]
        x_v[pl.ds(r0, _L)] = acc_ui - acc_uj + bi - bj
        swu_v[pl.ds(r0, _L)] = acc_wu
        shi_v[pl.ds(r0, _L)] = acc_hi
        shj_v[pl.ds(r0, _L)] = acc_hj
        bs_v[pl.ds(r0, _L)] = bi + bj
        return 0

    lax.fori_loop(0, _BPW // _L, block, 0)

    pltpu.sync_copy(x_v, x_hbm.at[pl.ds(base, _BPW)])
    pltpu.sync_copy(swu_v, swu_hbm.at[pl.ds(base, _BPW)])
    pltpu.sync_copy(shi_v, shi_hbm.at[pl.ds(base, _BPW)])
    pltpu.sync_copy(shj_v, shj_hbm.at[pl.ds(base, _BPW)])
    pltpu.sync_copy(bs_v, bs_hbm.at[pl.ds(base, _BPW)])


@jax.jit
def _sc_partials(u, i, j, W, H, B):
    f32 = jnp.float32
    mesh = plsc.VectorSubcoreMesh(core_axis_name="c", subcore_axis_name="s")
    out = pl.kernel(
        _sc_body,
        mesh=mesh,
        compiler_params=pltpu.CompilerParams(
            needs_layout_passes=False, use_tc_tiling_on_sc=False
        ),
        out_type=[jax.ShapeDtypeStruct((N,), f32) for _ in range(5)],
        scratch_types=[
            pltpu.VMEM((_BPW,), jnp.int32),
            pltpu.VMEM((_BPW,), jnp.int32),
            pltpu.VMEM((_BPW,), jnp.int32),
            pltpu.VMEM((_BPW, DIM), f32),
            pltpu.VMEM((_BPW, DIM), f32),
            pltpu.VMEM((_BPW, DIM), f32),
            pltpu.VMEM((_BPW,), f32),
            pltpu.VMEM((_BPW,), f32),
            pltpu.VMEM((_BPW,), f32),
            pltpu.VMEM((_BPW,), f32),
            pltpu.VMEM((_BPW,), f32),
            pltpu.VMEM((_BPW,), f32),
            pltpu.VMEM((_BPW,), f32),
            pltpu.SemaphoreType.DMA,
        ],
    )(u, i, j, W, H, B)
    return out


def _tc_body(x_ref, swu_ref, shi_ref, shj_ref, bs_ref, out_ref):
    x = x_ref[...]
    lp = jnp.mean(-jnp.log(1.0 + jnp.exp(-x)))
    lp = lp - REG_USER * jnp.mean(jnp.sqrt(swu_ref[...]))
    lp = lp - REG_POS_ITEM * jnp.mean(jnp.sqrt(shi_ref[...]))
    lp = lp - REG_NEG_ITEM * jnp.mean(jnp.sqrt(shj_ref[...]))
    lp = lp - REG_BIAS * jnp.mean(bs_ref[...])
    out_ref[0, 0] = -lp


@jax.jit
def _tc_reduce(x, swu, shi, shj, bs):
    r = lambda a: a.reshape(128, 128)
    out = pl.pallas_call(
        _tc_body,
        out_shape=jax.ShapeDtypeStruct((1, 1), jnp.float32),
        out_specs=pl.BlockSpec(memory_space=pltpu.SMEM),
    )(r(x), r(swu), r(shi), r(shj), r(bs))
    return out[0, 0]


def kernel(u, i, j, W, H, B):
    x, swu, shi, shj, bs = _sc_partials(u, i, j, W, H, B)
    return _tc_reduce(x, swu, shi, shj, bs)
